# Initial kernel scaffold; baseline (speedup 1.0000x reference)
import jax
import jax.numpy as jnp
from jax.experimental import pallas as pl


def kernel(inputs, paths, negs, target_table, context_table):
    raise NotImplementedError("write your pallas kernel here")


# speedup vs baseline: 3.5281x; 3.5281x over previous
import functools

import jax
import jax.numpy as jnp
import numpy as np
from jax import lax
from jax.experimental import pallas as pl
from jax.experimental.pallas import tpu as pltpu
from jax.experimental.pallas import tpu_sc as plsc

_MAX_ID = 99999
_DIM = 64
_WALK_LEN = 5
_WALK_NUM = 2
_WIN = 2
_NUM_NEGS = 5
_BATCH = 1024
_L = _WALK_LEN + 1

_PAIR_SRC = []
_PAIR_POS = []
for _i in range(_L):
    for _j in range(max(0, _i - _WIN), min(_L, _i + _WIN + 1)):
        if _j != _i:
            _PAIR_SRC.append(_i)
            _PAIR_POS.append(_j)
_NUM_PAIRS = len(_PAIR_SRC)

_NPAIRS_TOTAL = _WALK_NUM * _BATCH * _NUM_PAIRS

_NC = 2
_NS = 16
_NW = _NC * _NS

_PER_W = _NPAIRS_TOTAL // _NW
_BATCH_PER_W = _PER_W // _NUM_PAIRS
_CHUNK = 128
_NCHUNK = _PER_W // _CHUNK
_NEG_PER_W = _PER_W * _NUM_NEGS
_IN_PER_W = _BATCH // _NW

_OFF_S = np.array(
    [(j // _NUM_PAIRS) * _L + _PAIR_SRC[j % _NUM_PAIRS] for j in range(_PER_W)],
    dtype=np.int32)
_OFF_P = np.array(
    [(j // _NUM_PAIRS) * _L + _PAIR_POS[j % _NUM_PAIRS] for j in range(_PER_W)],
    dtype=np.int32)


def _sc_body(paths_ref, negs_ref, offs_ref, offp_ref, inputs_ref,
             tgt_ref, ctx_ref,
             part_ref, oemb_ref, octx_ref,
             paths_v, offs_v, offp_v, sidx_v, pidx_v, nidx_v,
             srows_v, prows_v, nrows_v, pout_v, iidx_v, irows_v, sem):
    wid = lax.axis_index("s") * _NC + lax.axis_index("c")

    pltpu.sync_copy(paths_ref.at[wid], paths_v)
    pltpu.sync_copy(negs_ref.at[wid], nidx_v)
    pltpu.sync_copy(offs_ref, offs_v)
    pltpu.sync_copy(offp_ref, offp_v)
    pltpu.sync_copy(inputs_ref.at[wid], iidx_v)

    pltpu.async_copy(tgt_ref.at[iidx_v], irows_v, sem).wait()
    pltpu.sync_copy(irows_v, oemb_ref.at[wid])
    pltpu.async_copy(ctx_ref.at[iidx_v], irows_v, sem).wait()
    pltpu.sync_copy(irows_v, octx_ref.at[wid])

    for t in range(_PER_W // 16):
        row, col = (t * 16) // _CHUNK, (t * 16) % _CHUNK
        o_s = offs_v[pl.ds(t * 16, 16)]
        o_p = offp_v[pl.ds(t * 16, 16)]
        sidx_v[row, pl.ds(col, 16)] = plsc.load_gather(paths_v, [o_s])
        pidx_v[row, pl.ds(col, 16)] = plsc.load_gather(paths_v, [o_p])

    for c in range(_NCHUNK):
        descs = [
            pltpu.async_copy(tgt_ref.at[sidx_v.at[c]], srows_v, sem),
            pltpu.async_copy(ctx_ref.at[pidx_v.at[c]], prows_v, sem),
        ]
        for q in range(_NUM_NEGS):
            descs.append(pltpu.async_copy(
                ctx_ref.at[nidx_v.at[c * _NUM_NEGS + q]],
                nrows_v.at[pl.ds(q * _CHUNK, _CHUNK), :], sem))
        for d in descs:
            d.wait()

        def body(jl, carry):
            s = [srows_v[jl, pl.ds(k * 16, 16)] for k in range(4)]
            p = [prows_v[jl, pl.ds(k * 16, 16)] for k in range(4)]
            acc = s[0] * p[0] + s[1] * p[1] + s[2] * p[2] + s[3] * p[3]
            pout_v[jl, pl.ds(_NUM_NEGS * 16, 16)] = acc
            for q in range(_NUM_NEGS):
                r = jl * _NUM_NEGS + q
                n = [nrows_v[r, pl.ds(k * 16, 16)] for k in range(4)]
                acc = s[0] * n[0] + s[1] * n[1] + s[2] * n[2] + s[3] * n[3]
                pout_v[jl, pl.ds(q * 16, 16)] = acc
            return carry

        lax.fori_loop(0, _CHUNK, body, 0)
        pltpu.sync_copy(pout_v, part_ref.at[wid, pl.ds(c * _CHUNK, _CHUNK), :])


def _sc_call(paths_w, negs_w, offs, offp, inputs_w, target_table, context_table):
    mesh = plsc.VectorSubcoreMesh(core_axis_name="c", subcore_axis_name="s",
                                  num_cores=_NC, num_subcores=_NS)
    f = pl.kernel(
        _sc_body,
        out_type=(
            jax.ShapeDtypeStruct((_NW, _PER_W, 6 * 16), jnp.float32),
            jax.ShapeDtypeStruct((_NW, _IN_PER_W, _DIM), jnp.float32),
            jax.ShapeDtypeStruct((_NW, _IN_PER_W, _DIM), jnp.float32),
        ),
        mesh=mesh,
        compiler_params=pltpu.CompilerParams(needs_layout_passes=False,
                                             use_tc_tiling_on_sc=False),
        scratch_types=[
            pltpu.VMEM((_BATCH_PER_W * _L,), jnp.int32),
            pltpu.VMEM((_PER_W,), jnp.int32),
            pltpu.VMEM((_PER_W,), jnp.int32),
            pltpu.VMEM((_NCHUNK, _CHUNK), jnp.int32),
            pltpu.VMEM((_NCHUNK, _CHUNK), jnp.int32),
            pltpu.VMEM((_NEG_PER_W // _CHUNK, _CHUNK), jnp.int32),
            pltpu.VMEM((_CHUNK, _DIM), jnp.float32),
            pltpu.VMEM((_CHUNK, _DIM), jnp.float32),
            pltpu.VMEM((_CHUNK * _NUM_NEGS, _DIM), jnp.float32),
            pltpu.VMEM((_CHUNK, 6 * 16), jnp.float32),
            pltpu.VMEM((_IN_PER_W,), jnp.int32),
            pltpu.VMEM((_IN_PER_W, _DIM), jnp.float32),
            pltpu.SemaphoreType.DMA,
        ],
    )
    return f(paths_w, negs_w, offs, offp, inputs_w, target_table, context_table)


_TC_BLK = 2304
_TC_GRID = _NPAIRS_TOTAL // _TC_BLK


def _tc_loss_body(p_ref, loss_ref, mrr_ref):
    i = pl.program_id(0)
    x = p_ref[...]
    r = lax.broadcasted_iota(jnp.int32, (6 * 16, 8), 0) // 16
    cidx = lax.broadcasted_iota(jnp.int32, (6 * 16, 8), 1)
    g = (r == cidx).astype(jnp.float32)
    logits = jnp.dot(x, g, preferred_element_type=jnp.float32)
    col = lax.broadcasted_iota(jnp.int32, logits.shape, 1)
    pos = jnp.broadcast_to(logits[:, 5:6], logits.shape)
    sp_in = jnp.where(col == 5, -logits, logits)
    sp = jnp.maximum(sp_in, 0.0) + jnp.log(1.0 + jnp.exp(-jnp.abs(sp_in)))
    loss_blk = jnp.sum(jnp.where(col < 6, sp, 0.0))
    cnt = jnp.sum(jnp.where(col < 5, (logits >= pos).astype(jnp.float32), 0.0),
                  axis=1)
    mrr_blk = jnp.sum(1.0 / (cnt + 1.0))

    @pl.when(i == 0)
    def _init():
        loss_ref[...] = jnp.zeros_like(loss_ref)
        mrr_ref[...] = jnp.zeros_like(mrr_ref)

    loss_ref[...] += jnp.reshape(loss_blk, (1, 1))
    mrr_ref[...] += jnp.reshape(mrr_blk, (1, 1))


def _tc_loss(partials):
    return pl.pallas_call(
        _tc_loss_body,
        grid=(_TC_GRID,),
        in_specs=[pl.BlockSpec((_TC_BLK, 6 * 16), lambda i: (i, 0))],
        out_specs=(pl.BlockSpec((1, 1), lambda i: (0, 0)),
                   pl.BlockSpec((1, 1), lambda i: (0, 0))),
        out_shape=(jax.ShapeDtypeStruct((1, 1), jnp.float32),
                   jax.ShapeDtypeStruct((1, 1), jnp.float32)),
    )(partials)


@jax.jit
def kernel(inputs, paths, negs, target_table, context_table):
    paths_w = jnp.reshape(paths.astype(jnp.int32), (_NW, _BATCH_PER_W * _L))
    negs_w = jnp.reshape(negs.astype(jnp.int32), (_NW, _NEG_PER_W // _CHUNK, _CHUNK))
    inputs_w = jnp.reshape(inputs.astype(jnp.int32), (_NW, _IN_PER_W))
    offs = jnp.asarray(_OFF_S)
    offp = jnp.asarray(_OFF_P)

    part, oemb, octx = _sc_call(paths_w, negs_w, offs, offp, inputs_w,
                                target_table, context_table)

    loss2d, mrr2d = _tc_loss(jnp.reshape(part, (_NPAIRS_TOTAL, 6 * 16)))
    loss = loss2d[0, 0]
    mrr = mrr2d[0, 0] / jnp.float32(_NPAIRS_TOTAL)

    out_embedding = jnp.reshape(oemb, (_BATCH, _DIM))
    out_context = jnp.reshape(octx, (_BATCH, _DIM))
    return (out_embedding, out_context, loss, mrr)


# speedup vs baseline: 3.8795x; 1.0996x over previous
import functools

import jax
import jax.numpy as jnp
import numpy as np
from jax import lax
from jax.experimental import pallas as pl
from jax.experimental.pallas import tpu as pltpu
from jax.experimental.pallas import tpu_sc as plsc

_MAX_ID = 99999
_DIM = 64
_WALK_LEN = 5
_WALK_NUM = 2
_WIN = 2
_NUM_NEGS = 5
_BATCH = 1024
_L = _WALK_LEN + 1

_PAIR_SRC = []
_PAIR_POS = []
for _i in range(_L):
    for _j in range(max(0, _i - _WIN), min(_L, _i + _WIN + 1)):
        if _j != _i:
            _PAIR_SRC.append(_i)
            _PAIR_POS.append(_j)
_NUM_PAIRS = len(_PAIR_SRC)

_NPAIRS_TOTAL = _WALK_NUM * _BATCH * _NUM_PAIRS
_PAIRS_PER_WALK = _BATCH * _NUM_PAIRS

_NC = 2
_NS = 16
_NW = _NC * _NS

_PER_W = _NPAIRS_TOTAL // _NW
_BATCH_PER_W = _PER_W // _NUM_PAIRS
_CHUNK = 64
_NCHUNK = _PER_W // _CHUNK
_IN_PER_W = _BATCH // _NW
_W_PER_WALK = _PAIRS_PER_WALK // _PER_W

_ROW_TAB = np.array([j // _NUM_PAIRS for j in range(_PER_W)], dtype=np.int32)
_COL_S = np.array([_PAIR_SRC[j % _NUM_PAIRS] for j in range(_PER_W)],
                  dtype=np.int32)
_COL_P = np.array([_PAIR_POS[j % _NUM_PAIRS] for j in range(_PER_W)],
                  dtype=np.int32)


def _sc_body(paths_ref, negs_ref, rows_ref, cols_s_ref, cols_p_ref,
             inputs_ref, tgt_ref, ctx_ref,
             part_ref, oemb_ref, octx_ref,
             paths_v, rows_v, colss_v, colsp_v, sidx_v, pidx_v, nidx_v,
             srows_v, prows_v, nrows_v, pout_v, iidx_v, irows_e, irows_c,
             gsem0, gsem1, osem0, osem1, isem):
    wid = lax.axis_index("s") * _NC + lax.axis_index("c")
    walk = wid // _W_PER_WALK
    wrow = (wid % _W_PER_WALK)
    gsems = (gsem0, gsem1)
    osems = (osem0, osem1)

    pltpu.sync_copy(
        paths_ref.at[walk, pl.ds(wrow * _BATCH_PER_W, _BATCH_PER_W), :],
        paths_v)
    pltpu.sync_copy(negs_ref.at[wid], nidx_v)
    pltpu.sync_copy(rows_ref, rows_v)
    pltpu.sync_copy(cols_s_ref, colss_v)
    pltpu.sync_copy(cols_p_ref, colsp_v)
    pltpu.sync_copy(inputs_ref.at[pl.ds(wid * _IN_PER_W, _IN_PER_W)], iidx_v)

    d_ie = pltpu.async_copy(tgt_ref.at[iidx_v], irows_e, isem)
    d_ic = pltpu.async_copy(ctx_ref.at[iidx_v], irows_c, isem)

    for t in range(_PER_W // 16):
        row, col = (t * 16) // _CHUNK, (t * 16) % _CHUNK
        r = rows_v[pl.ds(t * 16, 16)]
        c_s = colss_v[pl.ds(t * 16, 16)]
        c_p = colsp_v[pl.ds(t * 16, 16)]
        sidx_v[row, pl.ds(col, 16)] = plsc.load_gather(paths_v, [r, c_s])
        pidx_v[row, pl.ds(col, 16)] = plsc.load_gather(paths_v, [r, c_p])

    g_descs = [None, None]
    o_descs = [None, None]

    def start_gathers(c):
        b = c % 2
        g_descs[b] = [
            pltpu.async_copy(tgt_ref.at[sidx_v.at[c]], srows_v.at[b],
                             gsems[b]),
            pltpu.async_copy(ctx_ref.at[pidx_v.at[c]], prows_v.at[b],
                             gsems[b]),
        ] + [
            pltpu.async_copy(
                ctx_ref.at[nidx_v.at[c * _NUM_NEGS + q]],
                nrows_v.at[b, pl.ds(q * _CHUNK, _CHUNK), :], gsems[b])
            for q in range(_NUM_NEGS)
        ]

    start_gathers(0)
    for c in range(_NCHUNK):
        b = c % 2
        if c + 1 < _NCHUNK:
            start_gathers(c + 1)
        for d in g_descs[b]:
            d.wait()
        if o_descs[b] is not None:
            o_descs[b].wait()

        def body(jl, carry):
            s = [srows_v[b, jl, pl.ds(k * 16, 16)] for k in range(4)]
            p = [prows_v[b, jl, pl.ds(k * 16, 16)] for k in range(4)]
            acc = s[0] * p[0] + s[1] * p[1] + s[2] * p[2] + s[3] * p[3]
            pout_v[b, jl, pl.ds(_NUM_NEGS * 16, 16)] = acc
            for q in range(_NUM_NEGS):
                r = jl * _NUM_NEGS + q
                n = [nrows_v[b, r, pl.ds(k * 16, 16)] for k in range(4)]
                acc = s[0] * n[0] + s[1] * n[1] + s[2] * n[2] + s[3] * n[3]
                pout_v[b, jl, pl.ds(q * 16, 16)] = acc
            return carry

        lax.fori_loop(0, _CHUNK, body, 0)
        o_descs[b] = pltpu.async_copy(
            pout_v.at[b],
            part_ref.at[pl.ds(wid * _PER_W + c * _CHUNK, _CHUNK), :],
            osems[b])

    for b in (0, 1):
        if o_descs[b] is not None:
            o_descs[b].wait()

    d_ie.wait()
    pltpu.sync_copy(irows_e, oemb_ref.at[pl.ds(wid * _IN_PER_W, _IN_PER_W), :])
    d_ic.wait()
    pltpu.sync_copy(irows_c, octx_ref.at[pl.ds(wid * _IN_PER_W, _IN_PER_W), :])


def _sc_call(paths, negs, rows_t, cols_s, cols_p, inputs,
             target_table, context_table):
    mesh = plsc.VectorSubcoreMesh(core_axis_name="c", subcore_axis_name="s",
                                  num_cores=_NC, num_subcores=_NS)
    f = pl.kernel(
        _sc_body,
        out_type=(
            jax.ShapeDtypeStruct((_NPAIRS_TOTAL, 6 * 16), jnp.float32),
            jax.ShapeDtypeStruct((_BATCH, _DIM), jnp.float32),
            jax.ShapeDtypeStruct((_BATCH, _DIM), jnp.float32),
        ),
        mesh=mesh,
        compiler_params=pltpu.CompilerParams(needs_layout_passes=False,
                                             use_tc_tiling_on_sc=False),
        scratch_types=[
            pltpu.VMEM((_BATCH_PER_W, _L), jnp.int32),
            pltpu.VMEM((_PER_W,), jnp.int32),
            pltpu.VMEM((_PER_W,), jnp.int32),
            pltpu.VMEM((_PER_W,), jnp.int32),
            pltpu.VMEM((_NCHUNK, _CHUNK), jnp.int32),
            pltpu.VMEM((_NCHUNK, _CHUNK), jnp.int32),
            pltpu.VMEM((_PER_W * _NUM_NEGS // _CHUNK, _CHUNK), jnp.int32),
            pltpu.VMEM((2, _CHUNK, _DIM), jnp.float32),
            pltpu.VMEM((2, _CHUNK, _DIM), jnp.float32),
            pltpu.VMEM((2, _CHUNK * _NUM_NEGS, _DIM), jnp.float32),
            pltpu.VMEM((2, _CHUNK, 6 * 16), jnp.float32),
            pltpu.VMEM((_IN_PER_W,), jnp.int32),
            pltpu.VMEM((_IN_PER_W, _DIM), jnp.float32),
            pltpu.VMEM((_IN_PER_W, _DIM), jnp.float32),
            pltpu.SemaphoreType.DMA,
            pltpu.SemaphoreType.DMA,
            pltpu.SemaphoreType.DMA,
            pltpu.SemaphoreType.DMA,
            pltpu.SemaphoreType.DMA,
        ],
    )
    return f(paths, negs, rows_t, cols_s, cols_p, inputs,
             target_table, context_table)


_TC_BLK = 2304
_TC_GRID = _NPAIRS_TOTAL // _TC_BLK


def _tc_loss_body(p_ref, loss_ref, mrr_ref):
    i = pl.program_id(0)
    x = p_ref[...]
    r = lax.broadcasted_iota(jnp.int32, (6 * 16, 8), 0) // 16
    cidx = lax.broadcasted_iota(jnp.int32, (6 * 16, 8), 1)
    g = (r == cidx).astype(jnp.float32)
    logits = jnp.dot(x, g, preferred_element_type=jnp.float32)
    col = lax.broadcasted_iota(jnp.int32, logits.shape, 1)
    pos = jnp.broadcast_to(logits[:, 5:6], logits.shape)
    sp_in = jnp.where(col == 5, -logits, logits)
    sp = jnp.maximum(sp_in, 0.0) + jnp.log(1.0 + jnp.exp(-jnp.abs(sp_in)))
    loss_blk = jnp.sum(jnp.where(col < 6, sp, 0.0))
    cnt = jnp.sum(jnp.where(col < 5, (logits >= pos).astype(jnp.float32), 0.0),
                  axis=1)
    mrr_blk = jnp.sum(1.0 / (cnt + 1.0))

    @pl.when(i == 0)
    def _init():
        loss_ref[...] = jnp.zeros_like(loss_ref)
        mrr_ref[...] = jnp.zeros_like(mrr_ref)

    loss_ref[...] += jnp.reshape(loss_blk, (1, 1))
    mrr_ref[...] += jnp.reshape(mrr_blk, (1, 1))


def _tc_loss(partials):
    return pl.pallas_call(
        _tc_loss_body,
        grid=(_TC_GRID,),
        in_specs=[pl.BlockSpec((_TC_BLK, 6 * 16), lambda i: (i, 0))],
        out_specs=(pl.BlockSpec((1, 1), lambda i: (0, 0)),
                   pl.BlockSpec((1, 1), lambda i: (0, 0))),
        out_shape=(jax.ShapeDtypeStruct((1, 1), jnp.float32),
                   jax.ShapeDtypeStruct((1, 1), jnp.float32)),
    )(partials)


@jax.jit
def kernel(inputs, paths, negs, target_table, context_table):
    rows_t = jnp.asarray(_ROW_TAB)
    cols_s = jnp.asarray(_COL_S)
    cols_p = jnp.asarray(_COL_P)

    negs_w = jnp.reshape(negs.astype(jnp.int32),
                         (_NW, _PER_W * _NUM_NEGS // _CHUNK, _CHUNK))
    part, oemb, octx = _sc_call(paths.astype(jnp.int32), negs_w,
                                rows_t, cols_s, cols_p,
                                inputs.astype(jnp.int32),
                                target_table, context_table)

    loss2d, mrr2d = _tc_loss(part)
    loss = loss2d[0, 0]
    mrr = mrr2d[0, 0] / jnp.float32(_NPAIRS_TOTAL)

    return (oemb, octx, loss, mrr)
